# baseline (device time: 141952 ns/iter reference)
import jax
import jax.numpy as jnp
from jax import lax
from jax.experimental import pallas as pl
from jax.experimental.pallas import tpu as pltpu

P = 16
NSTREAM = 4
XLEAD = 3


def kernel(x, w_mat):
    m_full, k_per = x.shape
    _, n = w_mat.shape
    m_per = m_full // P
    nq = n // NSTREAM
    nr = NSTREAM // 2

    def body(x_ref, w_ref, out_ref, *scr):
        xbf, wbf, xrow = scr[0], scr[1], scr[2]
        rbufs = scr[3:3 + NSTREAM]
        abuf, atile = scr[3 + NSTREAM], scr[4 + NSTREAM]
        sems = scr[5 + NSTREAM:]
        rsems = sems[0:NSTREAM]
        fsends = sems[NSTREAM:2 * NSTREAM]
        xsend_sem, xrecv_sem, asend_sem, arecv_sem = sems[2 * NSTREAM:]

        d = lax.axis_index("i")
        left = jnp.mod(d - 1, P)
        right = jnp.mod(d + 1, P)
        sdev = [right] * nr + [left] * nr
        qorder = [q for pair in zip(range(nr), range(nr, NSTREAM))
                  for q in pair]

        barrier = pltpu.get_barrier_semaphore()
        for o in range(1, P):
            pl.semaphore_signal(barrier, inc=1,
                                device_id=(jnp.mod(d + o, P),),
                                device_id_type=pl.DeviceIdType.MESH)

        xbf[...] = x_ref[...].astype(jnp.bfloat16)
        wbf[...] = w_ref[...].astype(jnp.bfloat16)

        pl.semaphore_wait(barrier, P - 1)

        xdescs = {}

        def issue_x(o):
            t = jnp.mod(d + o, P)
            desc = pltpu.make_async_remote_copy(
                src_ref=xbf.at[pl.ds(t * m_per, m_per), :],
                dst_ref=xrow.at[o],
                send_sem=xsend_sem.at[o - 1],
                recv_sem=xrecv_sem.at[o - 1],
                device_id=(t,),
                device_id_type=pl.DeviceIdType.MESH,
            )
            desc.start()
            xdescs[o] = desc

        for s0 in range(XLEAD):
            for o in (s0 + 1, P - 1 - s0):
                if o not in xdescs:
                    issue_x(o)

        descs = [[] for _ in range(NSTREAM)]
        for q in qorder:
            desc = pltpu.make_async_remote_copy(
                src_ref=wbf.at[:, pl.ds(q * nq, nq)],
                dst_ref=rbufs[q].at[0],
                send_sem=fsends[q].at[0],
                recv_sem=rsems[q].at[0],
                device_id=(sdev[q],),
                device_id_type=pl.DeviceIdType.MESH,
            )
            desc.start()
            descs[q].append(desc)

        xown = xbf[pl.ds(d * m_per, m_per), :]
        accs = [
            jnp.dot(xown, wbf[:, q * nq:(q + 1) * nq],
                    preferred_element_type=jnp.float32)
            for q in range(NSTREAM)
        ]

        x_waited = set()
        for s in range(P - 1):
            if s + XLEAD <= P - 2:
                for o in (s + XLEAD + 1, P - 1 - s - XLEAD):
                    if o not in xdescs:
                        issue_x(o)
            for q in qorder:
                descs[q][s].wait_recv()
                if s < P - 2:
                    desc = pltpu.make_async_remote_copy(
                        src_ref=rbufs[q].at[s],
                        dst_ref=rbufs[q].at[s + 1],
                        send_sem=fsends[q].at[s + 1],
                        recv_sem=rsems[q].at[s + 1],
                        device_id=(sdev[q],),
                        device_id_type=pl.DeviceIdType.MESH,
                    )
                    desc.start()
                    descs[q].append(desc)

            for o in (s + 1, P - 1 - s):
                if o not in x_waited:
                    xdescs[o].wait_recv()
                    x_waited.add(o)
            for q in range(NSTREAM):
                xr = xrow[s + 1] if q < nr else xrow[P - 1 - s]
                accs[q] = accs[q] + jnp.dot(
                    xr, rbufs[q][s], preferred_element_type=jnp.float32)

        my_amax = jnp.max(jnp.stack([jnp.max(a) for a in accs]))
        my_amax = jnp.maximum(my_amax, 0.0)

        atile[...] = jnp.full((8, 128), my_amax, jnp.float32)
        abuf[0] = atile[...]
        adescs = []
        for o in range(1, P):
            desc = pltpu.make_async_remote_copy(
                src_ref=atile,
                dst_ref=abuf.at[o],
                send_sem=asend_sem.at[o - 1],
                recv_sem=arecv_sem.at[o - 1],
                device_id=(jnp.mod(d + o, P),),
                device_id_type=pl.DeviceIdType.MESH,
            )
            desc.start()
            adescs.append(desc)

        for group in [list(xdescs.values())] + descs:
            for desc in group:
                desc.wait_send()
        ys = [jnp.maximum(accs[q], 0.0) for q in range(NSTREAM)]

        for desc in adescs:
            desc.wait_send()
        for desc in adescs:
            desc.wait_recv()
        amax = jnp.max(abuf[...])

        scale = amax / 448.0
        inv_scale = 448.0 / amax
        for q in range(NSTREAM):
            qq = jnp.minimum(ys[q] * inv_scale, 448.0).astype(
                jnp.float8_e4m3fn)
            out_ref[:, q * nq:(q + 1) * nq] = qq.astype(jnp.float32) * scale

    scratch = [
        pltpu.VMEM((m_full, k_per), jnp.bfloat16),
        pltpu.VMEM((k_per, n), jnp.bfloat16),
        pltpu.VMEM((P, m_per, k_per), jnp.bfloat16),
    ]
    scratch += [pltpu.VMEM((P - 1, k_per, nq), jnp.bfloat16)
                for _ in range(NSTREAM)]
    scratch += [
        pltpu.VMEM((P, 8, 128), jnp.float32),
        pltpu.VMEM((8, 128), jnp.float32),
    ]
    scratch += [pltpu.SemaphoreType.DMA((P - 1,))
                for _ in range(2 * NSTREAM)]
    scratch += [pltpu.SemaphoreType.DMA((P - 1,))
                for _ in range(4)]

    return pl.pallas_call(
        body,
        out_shape=jax.ShapeDtypeStruct((m_per, n), jnp.float32),
        in_specs=[
            pl.BlockSpec(memory_space=pltpu.VMEM),
            pl.BlockSpec(memory_space=pltpu.VMEM),
        ],
        out_specs=pl.BlockSpec(memory_space=pltpu.VMEM),
        scratch_shapes=scratch,
        compiler_params=pltpu.CompilerParams(
            collective_id=0, vmem_limit_bytes=100 * 1024 * 1024
        ),
    )(x, w_mat)


# device time: 134617 ns/iter; 1.0545x vs baseline; 1.0545x over previous
import jax
import jax.numpy as jnp
from jax import lax
from jax.experimental import pallas as pl
from jax.experimental.pallas import tpu as pltpu

P = 16
NR = 8
NL = 7
NQ = 2


def kernel(x, w_mat):
    m_full, k_per = x.shape
    _, n = w_mat.shape
    m_per = m_full // P
    nh = n // NQ

    def body(x_ref, w_ref, out_ref, *scr):
        xbf, wbf, xrow = scr[0], scr[1], scr[2]
        rbufR = scr[3:3 + NQ]
        rbufL = scr[3 + NQ:3 + 2 * NQ]
        sbufL = scr[3 + 2 * NQ:3 + 3 * NQ]
        abuf, atile = scr[3 + 3 * NQ], scr[4 + 3 * NQ]
        sems = scr[5 + 3 * NQ:]
        rsemR = sems[0:NQ]
        fsendR = sems[NQ:2 * NQ]
        rsemL = sems[2 * NQ:3 * NQ]
        ssemL = sems[3 * NQ:4 * NQ]
        xsend_sem, xrecv_sem, asend_sem, arecv_sem = sems[4 * NQ:]

        d = lax.axis_index("i")
        left = jnp.mod(d - 1, P)
        right = jnp.mod(d + 1, P)

        barrier = pltpu.get_barrier_semaphore()
        for o in range(1, P):
            pl.semaphore_signal(barrier, inc=1,
                                device_id=(jnp.mod(d + o, P),),
                                device_id_type=pl.DeviceIdType.MESH)

        xbf[...] = x_ref[...].astype(jnp.bfloat16)
        wbf[...] = w_ref[...].astype(jnp.bfloat16)

        pl.semaphore_wait(barrier, P - 1)

        xdescs = {}
        for o in range(1, NR + 1):
            t = jnp.mod(d + o, P)
            desc = pltpu.make_async_remote_copy(
                src_ref=xbf.at[pl.ds(t * m_per, m_per), :],
                dst_ref=xrow.at[o],
                send_sem=xsend_sem.at[o - 1],
                recv_sem=xrecv_sem.at[o - 1],
                device_id=(t,),
                device_id_type=pl.DeviceIdType.MESH,
            )
            desc.start()
            xdescs[o] = desc

        def pdot(c, q):
            xs = xbf[pl.ds(jnp.mod(c, P) * m_per, m_per), :]
            return jnp.dot(xs, wbf[:, q * nh:(q + 1) * nh],
                           preferred_element_type=jnp.float32)

        xown = xbf[pl.ds(d * m_per, m_per), :]
        accs = [
            jnp.dot(xown, wbf[:, q * nh:(q + 1) * nh],
                    preferred_element_type=jnp.float32)
            for q in range(NQ)
        ]
        for q in range(NQ):
            sbufL[q][0] = pdot(d - NL, q).astype(jnp.bfloat16)

        for o in range(1, NR + 1):
            xdescs[o].wait_recv()

        descsR = [[] for _ in range(NQ)]
        for q in range(NQ):
            desc = pltpu.make_async_remote_copy(
                src_ref=wbf.at[:, pl.ds(q * nh, nh)],
                dst_ref=rbufR[q].at[0],
                send_sem=fsendR[q].at[0],
                recv_sem=rsemR[q].at[0],
                device_id=(right,),
                device_id_type=pl.DeviceIdType.MESH,
            )
            desc.start()
            descsR[q].append(desc)

        descsL = [[] for _ in range(NQ)]
        pnext = [None] * NQ
        for s in range(NR):
            if s < NL:
                for q in range(NQ):
                    desc = pltpu.make_async_remote_copy(
                        src_ref=sbufL[q].at[s % 2],
                        dst_ref=rbufL[q].at[s],
                        send_sem=ssemL[q].at[s % 2],
                        recv_sem=rsemL[q].at[s],
                        device_id=(left,),
                        device_id_type=pl.DeviceIdType.MESH,
                    )
                    desc.start()
                    descsL[q].append(desc)

            if s < NL - 1:
                pnext = [pdot(d - (NL - 1) + s, q) for q in range(NQ)]

            for q in range(NQ):
                descsR[q][s].wait_recv()
                if s < NR - 1:
                    desc = pltpu.make_async_remote_copy(
                        src_ref=rbufR[q].at[s],
                        dst_ref=rbufR[q].at[s + 1],
                        send_sem=fsendR[q].at[s + 1],
                        recv_sem=rsemR[q].at[s + 1],
                        device_id=(right,),
                        device_id_type=pl.DeviceIdType.MESH,
                    )
                    desc.start()
                    descsR[q].append(desc)

            if s < NL:
                for q in range(NQ):
                    descsL[q][s].wait_recv()
                    if s < NL - 1:
                        val = rbufL[q][s].astype(jnp.float32) + pnext[q]
                        if s >= 1:
                            descsL[q][s - 1].wait_send()
                        sbufL[q][(s + 1) % 2] = val.astype(jnp.bfloat16)
                    else:
                        accs[q] = accs[q] + rbufL[q][s].astype(jnp.float32)

            for q in range(NQ):
                accs[q] = accs[q] + jnp.dot(
                    xrow[s + 1], rbufR[q][s],
                    preferred_element_type=jnp.float32)

        my_amax = jnp.max(jnp.stack([jnp.max(a) for a in accs]))
        my_amax = jnp.maximum(my_amax, 0.0)

        atile[...] = jnp.full((8, 128), my_amax, jnp.float32)
        abuf[0] = atile[...]
        adescs = []
        for o in range(1, P):
            desc = pltpu.make_async_remote_copy(
                src_ref=atile,
                dst_ref=abuf.at[o],
                send_sem=asend_sem.at[o - 1],
                recv_sem=arecv_sem.at[o - 1],
                device_id=(jnp.mod(d + o, P),),
                device_id_type=pl.DeviceIdType.MESH,
            )
            desc.start()
            adescs.append(desc)

        for group in [list(xdescs.values())] + descsR:
            for desc in group:
                desc.wait_send()
        for q in range(NQ):
            for desc in descsL[q][NL - 2:]:
                desc.wait_send()
        ys = [jnp.maximum(accs[q], 0.0) for q in range(NQ)]

        for desc in adescs:
            desc.wait_send()
        for desc in adescs:
            desc.wait_recv()
        amax = jnp.max(abuf[...])

        scale = amax / 448.0
        inv_scale = 448.0 / amax
        for q in range(NQ):
            qq = jnp.minimum(ys[q] * inv_scale, 448.0).astype(
                jnp.float8_e4m3fn)
            out_ref[:, q * nh:(q + 1) * nh] = qq.astype(jnp.float32) * scale

    scratch = [
        pltpu.VMEM((m_full, k_per), jnp.bfloat16),
        pltpu.VMEM((k_per, n), jnp.bfloat16),
        pltpu.VMEM((NR + 1, m_per, k_per), jnp.bfloat16),
    ]
    scratch += [pltpu.VMEM((NR, k_per, nh), jnp.bfloat16)
                for _ in range(NQ)]
    scratch += [pltpu.VMEM((NL, m_per, nh), jnp.bfloat16)
                for _ in range(NQ)]
    scratch += [pltpu.VMEM((2, m_per, nh), jnp.bfloat16)
                for _ in range(NQ)]
    scratch += [
        pltpu.VMEM((P, 8, 128), jnp.float32),
        pltpu.VMEM((8, 128), jnp.float32),
    ]
    scratch += [pltpu.SemaphoreType.DMA((NR,)) for _ in range(2 * NQ)]
    scratch += [pltpu.SemaphoreType.DMA((NL,)) for _ in range(NQ)]
    scratch += [pltpu.SemaphoreType.DMA((2,)) for _ in range(NQ)]
    scratch += [
        pltpu.SemaphoreType.DMA((NR,)),
        pltpu.SemaphoreType.DMA((NR,)),
        pltpu.SemaphoreType.DMA((P - 1,)),
        pltpu.SemaphoreType.DMA((P - 1,)),
    ]

    return pl.pallas_call(
        body,
        out_shape=jax.ShapeDtypeStruct((m_per, n), jnp.float32),
        in_specs=[
            pl.BlockSpec(memory_space=pltpu.VMEM),
            pl.BlockSpec(memory_space=pltpu.VMEM),
        ],
        out_specs=pl.BlockSpec(memory_space=pltpu.VMEM),
        scratch_shapes=scratch,
        compiler_params=pltpu.CompilerParams(
            collective_id=0, vmem_limit_bytes=100 * 1024 * 1024
        ),
    )(x, w_mat)
